# Initial kernel scaffold; baseline (speedup 1.0000x reference)
import jax
import jax.numpy as jnp
from jax import lax
from jax.experimental import pallas as pl
from jax.experimental.pallas import tpu as pltpu

N = 50000
B = 512
F = 256
H = 32
GF = 103
RB = 400
NBLK = N // RB

NEG_BIG = -3.0e38


def _body(x_ref, b_ref, gf_ref, W1_ref, b1_ref, W2_ref, b2_ref,
          out_ref, s_sc, g_sc, m_sc, d_sc):
    p = pl.program_id(0)
    i = pl.program_id(1)

    ids = b_ref[0]
    iota = lax.broadcasted_iota(jnp.int32, (RB, B), 1)
    oh = ids == iota
    ohf = oh.astype(jnp.float32)

    row0 = i * RB

    @pl.when(jnp.logical_and(p == 0, i == 0))
    def _init_g():
        g_sc[...] = (jnp.dot(gf_ref[...], W1_ref[F:, :],
                             preferred_element_type=jnp.float32)
                     + b1_ref[...])

    @pl.when(p == 0)
    def _phase0():
        a = jnp.dot(x_ref[...], W1_ref[:F, :],
                    preferred_element_type=jnp.float32)
        grow = jnp.dot(ohf, g_sc[...], preferred_element_type=jnp.float32)
        h = jax.nn.softplus(a + grow)
        s = jnp.dot(h, W2_ref[...],
                    preferred_element_type=jnp.float32) + b2_ref[...]
        s_sc[pl.ds(row0, RB), :] = s
        sb = jnp.where(oh, s, -jnp.inf)
        bmax = jnp.max(sb, axis=0, keepdims=True)
        prev = jnp.where(i == 0, jnp.full((1, B), -jnp.inf, jnp.float32),
                         m_sc[...])
        m_sc[...] = jnp.maximum(prev, bmax)

    @pl.when(p == 1)
    def _phase1():
        s = s_sc[pl.ds(row0, RB), :]
        m = m_sc[...]
        m = jnp.where(m > NEG_BIG, m, 0.0)
        m_rows = jnp.sum(ohf * m, axis=1, keepdims=True)
        e = jnp.exp(s - m_rows)
        s_sc[pl.ds(row0, RB), :] = e
        db = jnp.sum(jnp.where(oh, e, 0.0), axis=0, keepdims=True)
        prev = jnp.where(i == 0, jnp.zeros((1, B), jnp.float32), d_sc[...])
        d_sc[...] = prev + db

    @pl.when(p == 2)
    def _phase2():
        e = s_sc[pl.ds(row0, RB), :]
        d_rows = jnp.sum(ohf * d_sc[...], axis=1, keepdims=True)
        out_ref[...] = e / (d_rows + 1e-16)


def kernel(x, batch, global_feat, W1, b1, W2, b2):
    batch3 = batch.astype(jnp.int32).reshape(NBLK, RB, 1)
    b1r = b1.reshape(1, H).astype(jnp.float32)
    b2r = b2.reshape(1, 1).astype(jnp.float32)

    grid = (3, NBLK)
    out = pl.pallas_call(
        _body,
        grid=grid,
        in_specs=[
            pl.BlockSpec((RB, F), lambda p, i: (jnp.where(p == 0, i, 0), 0)),
            pl.BlockSpec((1, RB, 1), lambda p, i: (i, 0, 0)),
            pl.BlockSpec((B, GF), lambda p, i: (0, 0)),
            pl.BlockSpec((F + GF, H), lambda p, i: (0, 0)),
            pl.BlockSpec((1, H), lambda p, i: (0, 0)),
            pl.BlockSpec((H, 1), lambda p, i: (0, 0)),
            pl.BlockSpec((1, 1), lambda p, i: (0, 0)),
        ],
        out_specs=pl.BlockSpec((RB, 1), lambda p, i: (i, 0)),
        out_shape=jax.ShapeDtypeStruct((N, 1), jnp.float32),
        scratch_shapes=[
            pltpu.VMEM((N, 1), jnp.float32),
            pltpu.VMEM((B, H), jnp.float32),
            pltpu.VMEM((1, B), jnp.float32),
            pltpu.VMEM((1, B), jnp.float32),
        ],
    )(x, batch3, global_feat, W1, b1r, W2, b2r)
    return out


# speedup vs baseline: 2.3217x; 2.3217x over previous
import jax
import jax.numpy as jnp
from jax import lax
from jax.experimental import pallas as pl
from jax.experimental.pallas import tpu as pltpu

N = 50000
B = 512
F = 256
H = 32
GF = 103
RB = 400
NBLK = N // RB

NEG_BIG = -3.0e38


def _body(x_ref, b_ref, gf_ref, W1_ref, b1_ref, W2_ref, b2_ref,
          out_ref, s_sc, g_sc, m_sc, d_sc):
    p = pl.program_id(0)
    i = pl.program_id(1)

    ids = b_ref[0]
    iota = lax.broadcasted_iota(jnp.int32, (RB, B), 1)
    oh = ids == iota
    ohf = oh.astype(jnp.float32)

    row0 = i * RB

    @pl.when(jnp.logical_and(p == 0, i == 0))
    def _init_g():
        g_sc[...] = (jnp.dot(gf_ref[...], W1_ref[F:, :],
                             preferred_element_type=jnp.float32)
                     + b1_ref[...])

    @pl.when(p == 0)
    def _phase0():
        a = jnp.dot(x_ref[...], W1_ref[:F, :],
                    preferred_element_type=jnp.float32)
        grow = jnp.dot(ohf, g_sc[...], preferred_element_type=jnp.float32)
        h = jax.nn.softplus(a + grow)
        s = jnp.dot(h, W2_ref[...],
                    preferred_element_type=jnp.float32) + b2_ref[...]
        s_sc[pl.ds(row0, RB), :] = s
        sb = jnp.where(oh, s, -jnp.inf)
        bmax = jnp.max(sb, axis=0, keepdims=True)
        prev = jnp.where(i == 0, jnp.full((1, B), -jnp.inf, jnp.float32),
                         m_sc[...])
        m_sc[...] = jnp.maximum(prev, bmax)

    @pl.when(p == 1)
    def _phase1():
        s = s_sc[pl.ds(row0, RB), :]
        m = m_sc[...]
        m = jnp.where(m > NEG_BIG, m, 0.0)
        m_rows = jnp.sum(ohf * m, axis=1, keepdims=True)
        e = jnp.exp(s - m_rows)
        s_sc[pl.ds(row0, RB), :] = e
        db = jnp.sum(jnp.where(oh, e, 0.0), axis=0, keepdims=True)
        prev = jnp.where(i == 0, jnp.zeros((1, B), jnp.float32), d_sc[...])
        d_sc[...] = prev + db

    @pl.when(p == 2)
    def _phase2():
        e = s_sc[pl.ds(row0, RB), :]
        d_rows = jnp.sum(ohf * d_sc[...], axis=1, keepdims=True)
        out_ref[...] = e / (d_rows + 1e-16)


def kernel(x, batch, global_feat, W1, b1, W2, b2):
    batch3 = batch.astype(jnp.int32).reshape(NBLK, RB, 1)
    b1r = b1.reshape(1, H).astype(jnp.float32)
    b2r = b2.reshape(1, 1).astype(jnp.float32)

    grid = (3, NBLK)
    out = pl.pallas_call(
        _body,
        grid=grid,
        in_specs=[
            pl.BlockSpec((RB, F), lambda p, i: (jnp.where(p == 0, i, 0), 0)),
            pl.BlockSpec((1, RB, 1), lambda p, i: (i, 0, 0)),
            pl.BlockSpec((B, GF), lambda p, i: (0, 0)),
            pl.BlockSpec((F + GF, H), lambda p, i: (0, 0)),
            pl.BlockSpec((1, H), lambda p, i: (0, 0)),
            pl.BlockSpec((H, 1), lambda p, i: (0, 0)),
            pl.BlockSpec((1, 1), lambda p, i: (0, 0)),
        ],
        out_specs=pl.BlockSpec((RB, 1), lambda p, i: (jnp.where(p == 2, i, 0), 0)),
        out_shape=jax.ShapeDtypeStruct((N, 1), jnp.float32),
        scratch_shapes=[
            pltpu.VMEM((N, 1), jnp.float32),
            pltpu.VMEM((B, H), jnp.float32),
            pltpu.VMEM((1, B), jnp.float32),
            pltpu.VMEM((1, B), jnp.float32),
        ],
    )(x, batch3, global_feat, W1, b1r, W2, b2r)
    return out


# speedup vs baseline: 3.8625x; 1.6636x over previous
import functools

import jax
import jax.numpy as jnp
from jax import lax
from jax.experimental import pallas as pl
from jax.experimental.pallas import tpu as pltpu
from jax.experimental.pallas import tpu_sc as plsc

N = 50000
B = 512
F = 256
H = 32
GFD = 103
RB = 400
NBLK = N // RB
TB = 528
NC = 2
NS = 16
NW = NC * NS
NPAD = 50176
PW = NPAD // NW
VPW = PW // 16
L = 16

NEG_BIG = -3.0e38


def _tc_body(x_ref, b_ref, gf_ref, W1_ref, b1_ref, W2_ref, b2_ref,
             s_ref, m_ref, g_sc, m_sc):
    i = pl.program_id(0)

    ids = b_ref[0]
    iota = lax.broadcasted_iota(jnp.int32, (RB, TB), 1)
    oh = ids == iota
    ohf = oh.astype(jnp.float32)

    @pl.when(i == 0)
    def _init_g():
        g = (jnp.dot(gf_ref[...], W1_ref[F:, :],
                     preferred_element_type=jnp.float32) + b1_ref[...])
        g_sc[...] = jnp.concatenate(
            [g, jnp.zeros((TB - B, H), jnp.float32)], axis=0)

    a = jnp.dot(x_ref[...], W1_ref[:F, :], preferred_element_type=jnp.float32)
    grow = jnp.dot(ohf, g_sc[...], preferred_element_type=jnp.float32)
    h = jax.nn.softplus(a + grow)
    s = jnp.dot(h, W2_ref[...], preferred_element_type=jnp.float32) + b2_ref[...]
    s_ref[...] = s

    sb = jnp.where(oh, s, -jnp.inf)
    bmax = jnp.max(sb, axis=0, keepdims=True)
    prev = jnp.where(i == 0, jnp.full((1, TB), -jnp.inf, jnp.float32),
                     m_sc[...])
    newm = jnp.maximum(prev, bmax)
    m_sc[...] = newm
    m_ref[...] = jnp.where(newm > NEG_BIG, newm, 0.0)


def _tc_scores(x, batch3, global_feat, W1, b1r, W2, b2r):
    return pl.pallas_call(
        _tc_body,
        grid=(NBLK,),
        in_specs=[
            pl.BlockSpec((RB, F), lambda i: (i, 0)),
            pl.BlockSpec((1, RB, 1), lambda i: (i, 0, 0)),
            pl.BlockSpec((B, GFD), lambda i: (0, 0)),
            pl.BlockSpec((F + GFD, H), lambda i: (0, 0)),
            pl.BlockSpec((1, H), lambda i: (0, 0)),
            pl.BlockSpec((H, 1), lambda i: (0, 0)),
            pl.BlockSpec((1, 1), lambda i: (0, 0)),
        ],
        out_specs=[
            pl.BlockSpec((RB, 1), lambda i: (i, 0)),
            pl.BlockSpec((1, TB), lambda i: (0, 0)),
        ],
        out_shape=[
            jax.ShapeDtypeStruct((N, 1), jnp.float32),
            jax.ShapeDtypeStruct((1, TB), jnp.float32),
        ],
        scratch_shapes=[
            pltpu.VMEM((TB, H), jnp.float32),
            pltpu.VMEM((1, TB), jnp.float32),
        ],
    )(x, batch3, global_feat, W1, b1r, W2, b2r)


_SC_MESH = plsc.VectorSubcoreMesh(core_axis_name="c", subcore_axis_name="s")
_SC_PARAMS = pltpu.CompilerParams(needs_layout_passes=False)


def _worker_id():
    return lax.axis_index("s") * NC + lax.axis_index("c")


@functools.partial(
    pl.kernel,
    out_type=[
        jax.ShapeDtypeStruct((NPAD,), jnp.float32),
        jax.ShapeDtypeStruct((NW, TB), jnp.float32),
    ],
    mesh=_SC_MESH,
    scratch_types=[
        pltpu.VMEM((PW,), jnp.float32),
        pltpu.VMEM((PW,), jnp.int32),
        pltpu.VMEM((PW,), jnp.float32),
        pltpu.VMEM((TB,), jnp.float32),
        pltpu.VMEM((TB,), jnp.float32),
        pltpu.VMEM((L,), jnp.int32),
        pltpu.VMEM((L,), jnp.float32),
    ],
    compiler_params=_SC_PARAMS,
)
def _sc_eden(s_hbm, ids_hbm, m_hbm, e_hbm, dp_hbm,
             s_v, id_v, e_v, m_v, d_v, bufi, bufv):
    wid = _worker_id()
    base = wid * PW
    pltpu.sync_copy(s_hbm.at[pl.ds(base, PW)], s_v)
    pltpu.sync_copy(ids_hbm.at[pl.ds(base, PW)], id_v)
    pltpu.sync_copy(m_hbm, m_v)

    zero16 = jnp.zeros((L,), jnp.float32)
    for j in range(TB // L):
        d_v[pl.ds(j * L, L)] = zero16

    lane = lax.iota(jnp.int32, L)
    shifts = [(k, jnp.maximum(lane - k, 0)) for k in (1, 2, 4, 8)]
    inx = jnp.minimum(lane + 1, L - 1)

    def it(v, carry):
        off = v * L
        ids = id_v[pl.ds(off, L)]
        sv = s_v[pl.ds(off, L)]
        mg = plsc.load_gather(m_v, [ids])
        e = jnp.exp(sv - mg)
        e_v[pl.ds(off, L)] = e
        bufi[...] = ids
        cur = e
        for k, idxk in shifts:
            ids_sh = plsc.load_gather(bufi, [idxk])
            bufv[...] = cur
            cur_sh = plsc.load_gather(bufv, [idxk])
            ok = jnp.logical_and(ids_sh == ids, lane >= k)
            cur = jnp.where(ok, cur + cur_sh, cur)
        ids_nx = plsc.load_gather(bufi, [inx])
        last = jnp.logical_or(ids_nx != ids, lane == L - 1)
        dg = plsc.load_gather(d_v, [ids])
        plsc.store_scatter(d_v, [ids], dg + cur, mask=last)
        return carry

    lax.fori_loop(0, VPW, it, 0)

    pltpu.sync_copy(e_v, e_hbm.at[pl.ds(base, PW)])
    pltpu.sync_copy(d_v, dp_hbm.at[wid])


@functools.partial(
    pl.kernel,
    out_type=jax.ShapeDtypeStruct((NPAD,), jnp.float32),
    mesh=_SC_MESH,
    scratch_types=[
        pltpu.VMEM((PW,), jnp.float32),
        pltpu.VMEM((PW,), jnp.int32),
        pltpu.VMEM((NW, TB), jnp.float32),
        pltpu.VMEM((TB,), jnp.float32),
    ],
    compiler_params=_SC_PARAMS,
)
def _sc_norm(e_hbm, ids_hbm, dp_hbm, w_hbm, e_v, id_v, dall_v, d_v):
    wid = _worker_id()
    base = wid * PW
    pltpu.sync_copy(e_hbm.at[pl.ds(base, PW)], e_v)
    pltpu.sync_copy(ids_hbm.at[pl.ds(base, PW)], id_v)
    pltpu.sync_copy(dp_hbm, dall_v)

    def jt(j, carry):
        off = j * L
        acc = jnp.zeros((L,), jnp.float32)
        for r in range(NW):
            acc = acc + dall_v[r, pl.ds(off, L)]
        d_v[pl.ds(off, L)] = acc
        return carry

    lax.fori_loop(0, TB // L, jt, 0)

    def it(v, carry):
        off = v * L
        ids = id_v[pl.ds(off, L)]
        e = e_v[pl.ds(off, L)]
        dg = plsc.load_gather(d_v, [ids])
        e_v[pl.ds(off, L)] = e / (dg + 1e-16)
        return carry

    lax.fori_loop(0, VPW, it, 0)

    pltpu.sync_copy(e_v, w_hbm.at[pl.ds(base, PW)])


def kernel(x, batch, global_feat, W1, b1, W2, b2):
    batch_i32 = batch.astype(jnp.int32)
    batch3 = batch_i32.reshape(NBLK, RB, 1)
    b1r = b1.reshape(1, H).astype(jnp.float32)
    b2r = b2.reshape(1, 1).astype(jnp.float32)

    s, mseg = _tc_scores(x, batch3, global_feat, W1, b1r, W2, b2r)

    s_pad = jnp.pad(s.reshape(N), (0, NPAD - N))
    ids_pad = jnp.pad(batch_i32, (0, NPAD - N), constant_values=B)
    e_pad, dparts = _sc_eden(s_pad, ids_pad, mseg.reshape(TB))
    w_pad = _sc_norm(e_pad, ids_pad, dparts)
    return w_pad[:N].reshape(N, 1)


# speedup vs baseline: 4.0588x; 1.0508x over previous
import functools

import jax
import jax.numpy as jnp
from jax import lax
from jax.experimental import pallas as pl
from jax.experimental.pallas import tpu as pltpu
from jax.experimental.pallas import tpu_sc as plsc

N = 50000
B = 512
F = 256
H = 32
GFD = 103
TB = 528
NS = 16
NPAD = 50176
TPW = NPAD // NS
RB = 448
NBLK = NPAD // RB
L = 16

NEG_BIG = -3.0e38


def _k3_body(x_ref, b_ref, gf_ref, W1_ref, b1_ref, W2_ref, b2_ref,
             s_ref, g_sc):
    i = pl.program_id(0)

    @pl.when(i == 0)
    def _init_g():
        g = (jnp.dot(gf_ref[...], W1_ref[F:, :],
                     preferred_element_type=jnp.float32) + b1_ref[...])
        g_sc[...] = jnp.concatenate(
            [g, jnp.zeros((TB - B, H), jnp.float32)], axis=0)

    ids = b_ref[0]
    iota = lax.broadcasted_iota(jnp.int32, (RB, TB), 1)
    ohf = (ids == iota).astype(jnp.float32)

    a = jnp.dot(x_ref[...], W1_ref[:F, :], preferred_element_type=jnp.float32)
    grow = jnp.dot(ohf, g_sc[...], preferred_element_type=jnp.float32)
    h = jax.nn.softplus(a + grow)
    s_ref[...] = (jnp.dot(h, W2_ref[...],
                          preferred_element_type=jnp.float32) + b2_ref[...])


def _k3_scores(x, batch3, global_feat, W1, b1r, W2, b2r):
    return pl.pallas_call(
        _k3_body,
        grid=(NBLK,),
        in_specs=[
            pl.BlockSpec((RB, F), lambda i: (i, 0)),
            pl.BlockSpec((1, RB, 1), lambda i: (i, 0, 0)),
            pl.BlockSpec((B, GFD), lambda i: (0, 0)),
            pl.BlockSpec((F + GFD, H), lambda i: (0, 0)),
            pl.BlockSpec((1, H), lambda i: (0, 0)),
            pl.BlockSpec((H, 1), lambda i: (0, 0)),
            pl.BlockSpec((1, 1), lambda i: (0, 0)),
        ],
        out_specs=pl.BlockSpec((RB, 1), lambda i: (i, 0)),
        out_shape=jax.ShapeDtypeStruct((NPAD, 1), jnp.float32),
        scratch_shapes=[
            pltpu.VMEM((TB, H), jnp.float32),
        ],
    )(x, batch3, global_feat, W1, b1r, W2, b2r)


_SC_MESH = plsc.VectorSubcoreMesh(core_axis_name="c", subcore_axis_name="s")
_SC_PARAMS = pltpu.CompilerParams(needs_layout_passes=False)


@functools.partial(
    pl.kernel,
    out_type=jax.ShapeDtypeStruct((NPAD,), jnp.float32),
    mesh=_SC_MESH,
    scratch_types=[
        pltpu.VMEM((TPW,), jnp.float32),
        pltpu.VMEM((TPW,), jnp.int32),
        pltpu.VMEM((TPW,), jnp.float32),
        pltpu.VMEM((TB,), jnp.float32),
        pltpu.VMEM((TB,), jnp.float32),
        pltpu.VMEM((NS, TB), jnp.float32),
        pltpu.VMEM_SHARED((NS, TB), jnp.float32),
        pltpu.VMEM_SHARED((NS, TB), jnp.float32),
        pltpu.VMEM((L,), jnp.int32),
        pltpu.VMEM((L,), jnp.float32),
    ],
    compiler_params=_SC_PARAMS,
)
def _k4_softmax(s_hbm, ids_hbm, w_hbm,
                s_v, id_v, e_v, mtab, dtab, allv, shm, shd, bufi, bufv):
    cid = lax.axis_index("c")
    sid = lax.axis_index("s")

    @pl.when(cid == 0)
    def _core0():
        base = sid * TPW
        pltpu.sync_copy(s_hbm.at[pl.ds(base, TPW)], s_v)
        pltpu.sync_copy(ids_hbm.at[pl.ds(base, TPW)], id_v)

        lane = lax.iota(jnp.int32, L)
        shifts = [(k, jnp.maximum(lane - k, 0)) for k in (1, 2, 4, 8)]
        inx = jnp.minimum(lane + 1, L - 1)
        nvr = TPW // L

        for j in range(TB // L):
            mtab[pl.ds(j * L, L)] = jnp.full((L,), NEG_BIG, jnp.float32)

        def scan_combine(ids, val, op):
            bufi[...] = ids
            cur = val
            for k, idxk in shifts:
                ids_sh = plsc.load_gather(bufi, [idxk])
                bufv[...] = cur
                cur_sh = plsc.load_gather(bufv, [idxk])
                ok = jnp.logical_and(ids_sh == ids, lane >= k)
                cur = jnp.where(ok, op(cur, cur_sh), cur)
            ids_nx = plsc.load_gather(bufi, [inx])
            last = jnp.logical_or(ids_nx != ids, lane == L - 1)
            return cur, last

        def max_it(v, carry):
            off = v * L
            ids = id_v[pl.ds(off, L)]
            sv = s_v[pl.ds(off, L)]
            cur, last = scan_combine(ids, sv, jnp.maximum)
            mg = plsc.load_gather(mtab, [ids])
            plsc.store_scatter(mtab, [ids], jnp.maximum(mg, cur), mask=last)
            return carry

        lax.fori_loop(0, nvr, max_it, 0)

        pltpu.sync_copy(mtab, shm.at[sid])
        plsc.subcore_barrier()
        pltpu.sync_copy(shm, allv)

        def mred(j, carry):
            off = j * L
            acc = jnp.full((L,), NEG_BIG, jnp.float32)
            for r in range(NS):
                acc = jnp.maximum(acc, allv[r, pl.ds(off, L)])
            mtab[pl.ds(off, L)] = jnp.where(acc > NEG_BIG, acc, 0.0)
            dtab[pl.ds(off, L)] = jnp.zeros((L,), jnp.float32)
            return carry

        lax.fori_loop(0, TB // L, mred, 0)

        def sum_it(v, carry):
            off = v * L
            ids = id_v[pl.ds(off, L)]
            sv = s_v[pl.ds(off, L)]
            mg = plsc.load_gather(mtab, [ids])
            e = jnp.exp(sv - mg)
            e_v[pl.ds(off, L)] = e
            cur, last = scan_combine(ids, e, jnp.add)
            dg = plsc.load_gather(dtab, [ids])
            plsc.store_scatter(dtab, [ids], dg + cur, mask=last)
            return carry

        lax.fori_loop(0, nvr, sum_it, 0)

        pltpu.sync_copy(dtab, shd.at[sid])
        plsc.subcore_barrier()
        pltpu.sync_copy(shd, allv)

        def dred(j, carry):
            off = j * L
            acc = jnp.zeros((L,), jnp.float32)
            for r in range(NS):
                acc = acc + allv[r, pl.ds(off, L)]
            dtab[pl.ds(off, L)] = acc
            return carry

        lax.fori_loop(0, TB // L, dred, 0)

        def norm_it(v, carry):
            off = v * L
            ids = id_v[pl.ds(off, L)]
            dg = plsc.load_gather(dtab, [ids])
            e_v[pl.ds(off, L)] = e_v[pl.ds(off, L)] / (dg + 1e-16)
            return carry

        lax.fori_loop(0, nvr, norm_it, 0)

        pltpu.sync_copy(e_v, w_hbm.at[pl.ds(base, TPW)])


def kernel(x, batch, global_feat, W1, b1, W2, b2):
    batch_i32 = batch.astype(jnp.int32)
    ids_pad = jnp.pad(batch_i32, (0, NPAD - N), constant_values=B)
    batch3 = ids_pad.reshape(NBLK, RB, 1)
    b1r = b1.reshape(1, H).astype(jnp.float32)
    b2r = b2.reshape(1, 1).astype(jnp.float32)

    s = _k3_scores(x, batch3, global_feat, W1, b1r, W2, b2r)
    w_pad = _k4_softmax(s.reshape(NPAD), ids_pad)
    return w_pad[:N].reshape(N, 1)


# speedup vs baseline: 5.9695x; 1.4708x over previous
import functools

import jax
import jax.numpy as jnp
from jax import lax
from jax.experimental import pallas as pl
from jax.experimental.pallas import tpu as pltpu
from jax.experimental.pallas import tpu_sc as plsc

N = 50000
B = 512
F = 256
H = 32
GFD = 103
TB = 528
NS = 16
NPAD = 50176
TPW = NPAD // NS
RB = 512
NBLK = NPAD // RB
L = 16

NEG_BIG = -3.0e38


def _k3_body(x_ref, b_ref, gf_ref, W1_ref, b1_ref, W2_ref, b2_ref,
             s_ref, g_sc):
    i = pl.program_id(0)

    @pl.when(i == 0)
    def _init_g():
        g = (jnp.dot(gf_ref[...], W1_ref[F:, :],
                     preferred_element_type=jnp.float32) + b1_ref[...])
        g_sc[...] = jnp.concatenate(
            [g, jnp.zeros((TB - B, H), jnp.float32)], axis=0)

    ids = b_ref[...].reshape(1, RB)
    iota = lax.broadcasted_iota(jnp.int32, (TB, RB), 0)
    ohf = (ids == iota).astype(jnp.float32)

    a = jnp.dot(x_ref[...], W1_ref[:F, :], preferred_element_type=jnp.float32)
    grow = lax.dot_general(ohf, g_sc[...], (((0,), (0,)), ((), ())),
                           preferred_element_type=jnp.float32)
    h = jax.nn.softplus(a + grow)
    srow = lax.dot_general(W2_ref[...], h, (((0,), (1,)), ((), ())),
                           preferred_element_type=jnp.float32)
    s_ref[...] = (srow + b2_ref[...])[0]


def _k3_scores(x, ids_pad, global_feat, W1, b1r, W2, b2r):
    return pl.pallas_call(
        _k3_body,
        grid=(NBLK,),
        in_specs=[
            pl.BlockSpec((RB, F), lambda i: (i, 0)),
            pl.BlockSpec((RB,), lambda i: (i,)),
            pl.BlockSpec((B, GFD), lambda i: (0, 0)),
            pl.BlockSpec((F + GFD, H), lambda i: (0, 0)),
            pl.BlockSpec((1, H), lambda i: (0, 0)),
            pl.BlockSpec((H, 1), lambda i: (0, 0)),
            pl.BlockSpec((1, 1), lambda i: (0, 0)),
        ],
        out_specs=pl.BlockSpec((RB,), lambda i: (i,)),
        out_shape=jax.ShapeDtypeStruct((NPAD,), jnp.float32),
        scratch_shapes=[
            pltpu.VMEM((TB, H), jnp.float32),
        ],
    )(x, ids_pad, global_feat, W1, b1r, W2, b2r)


_SC_MESH = plsc.VectorSubcoreMesh(core_axis_name="c", subcore_axis_name="s")
_SC_PARAMS = pltpu.CompilerParams(needs_layout_passes=False)


@functools.partial(
    pl.kernel,
    out_type=jax.ShapeDtypeStruct((NPAD,), jnp.float32),
    mesh=_SC_MESH,
    scratch_types=[
        pltpu.VMEM((TPW,), jnp.float32),
        pltpu.VMEM((TPW,), jnp.int32),
        pltpu.VMEM((TPW,), jnp.float32),
        pltpu.VMEM((TB,), jnp.float32),
        pltpu.VMEM((TB,), jnp.float32),
        pltpu.VMEM((NS, TB), jnp.float32),
        pltpu.VMEM_SHARED((NS, TB), jnp.float32),
        pltpu.VMEM_SHARED((NS, TB), jnp.float32),
        pltpu.VMEM((L,), jnp.int32),
        pltpu.VMEM((L,), jnp.float32),
    ],
    compiler_params=_SC_PARAMS,
)
def _k4_softmax(s_hbm, ids_hbm, w_hbm,
                s_v, id_v, e_v, mtab, dtab, allv, shm, shd, bufi, bufv):
    cid = lax.axis_index("c")
    sid = lax.axis_index("s")

    @pl.when(cid == 0)
    def _core0():
        base = sid * TPW
        pltpu.sync_copy(s_hbm.at[pl.ds(base, TPW)], s_v)
        pltpu.sync_copy(ids_hbm.at[pl.ds(base, TPW)], id_v)

        lane = lax.iota(jnp.int32, L)
        shifts = [(k, jnp.maximum(lane - k, 0)) for k in (1, 2, 4, 8)]
        inx = jnp.minimum(lane + 1, L - 1)
        nvr = TPW // L

        for j in range(TB // L):
            mtab[pl.ds(j * L, L)] = jnp.full((L,), NEG_BIG, jnp.float32)

        def scan_combine(ids, val, op):
            bufi[...] = ids
            cur = val
            for k, idxk in shifts:
                ids_sh = plsc.load_gather(bufi, [idxk])
                bufv[...] = cur
                cur_sh = plsc.load_gather(bufv, [idxk])
                ok = jnp.logical_and(ids_sh == ids, lane >= k)
                cur = jnp.where(ok, op(cur, cur_sh), cur)
            ids_nx = plsc.load_gather(bufi, [inx])
            last = jnp.logical_or(ids_nx != ids, lane == L - 1)
            return cur, last

        def max_it(v, carry):
            off = v * L
            ids = id_v[pl.ds(off, L)]
            sv = s_v[pl.ds(off, L)]
            cur, last = scan_combine(ids, sv, jnp.maximum)
            mg = plsc.load_gather(mtab, [ids])
            plsc.store_scatter(mtab, [ids], jnp.maximum(mg, cur), mask=last)
            return carry

        lax.fori_loop(0, nvr, max_it, 0)

        pltpu.sync_copy(mtab, shm.at[sid])
        plsc.subcore_barrier()
        pltpu.sync_copy(shm, allv)

        def mred(j, carry):
            off = j * L
            acc = jnp.full((L,), NEG_BIG, jnp.float32)
            for r in range(NS):
                acc = jnp.maximum(acc, allv[r, pl.ds(off, L)])
            mtab[pl.ds(off, L)] = jnp.where(acc > NEG_BIG, acc, 0.0)
            dtab[pl.ds(off, L)] = jnp.zeros((L,), jnp.float32)
            return carry

        lax.fori_loop(0, TB // L, mred, 0)

        def sum_it(v, carry):
            off = v * L
            ids = id_v[pl.ds(off, L)]
            sv = s_v[pl.ds(off, L)]
            mg = plsc.load_gather(mtab, [ids])
            e = jnp.exp(sv - mg)
            e_v[pl.ds(off, L)] = e
            cur, last = scan_combine(ids, e, jnp.add)
            dg = plsc.load_gather(dtab, [ids])
            plsc.store_scatter(dtab, [ids], dg + cur, mask=last)
            return carry

        lax.fori_loop(0, nvr, sum_it, 0)

        pltpu.sync_copy(dtab, shd.at[sid])
        plsc.subcore_barrier()
        pltpu.sync_copy(shd, allv)

        def dred(j, carry):
            off = j * L
            acc = jnp.zeros((L,), jnp.float32)
            for r in range(NS):
                acc = acc + allv[r, pl.ds(off, L)]
            dtab[pl.ds(off, L)] = acc
            return carry

        lax.fori_loop(0, TB // L, dred, 0)

        def norm_it(v, carry):
            off = v * L
            ids = id_v[pl.ds(off, L)]
            dg = plsc.load_gather(dtab, [ids])
            e_v[pl.ds(off, L)] = e_v[pl.ds(off, L)] / (dg + 1e-16)
            return carry

        lax.fori_loop(0, nvr, norm_it, 0)

        pltpu.sync_copy(e_v, w_hbm.at[pl.ds(base, TPW)])


def kernel(x, batch, global_feat, W1, b1, W2, b2):
    batch_i32 = batch.astype(jnp.int32)
    ids_pad = jnp.pad(batch_i32, (0, NPAD - N), constant_values=B)
    b1r = b1.reshape(1, H).astype(jnp.float32)
    b2r = b2.reshape(1, 1).astype(jnp.float32)

    s = _k3_scores(x, ids_pad, global_feat, W1, b1r, W2, b2r)
    w_pad = _k4_softmax(s, ids_pad)
    return w_pad[:N].reshape(N, 1)


# speedup vs baseline: 7.6849x; 1.2874x over previous
import functools

import jax
import jax.numpy as jnp
from jax import lax
from jax.experimental import pallas as pl
from jax.experimental.pallas import tpu as pltpu
from jax.experimental.pallas import tpu_sc as plsc

N = 50000
B = 512
F = 256
H = 32
GFD = 103
TB = 528
NS = 16
NPAD = 50176
TPW = NPAD // NS
RB = 1024
NBLK = NPAD // RB
L = 16

NEG_BIG = -3.0e38


def _k3_body(x_ref, b_ref, gf_ref, W1_ref, b1_ref, W2_ref, b2_ref,
             s_ref, g_sc):
    i = pl.program_id(0)

    @pl.when(i == 0)
    def _init_g():
        g = (jnp.dot(gf_ref[...], W1_ref[F:, :],
                     preferred_element_type=jnp.float32) + b1_ref[...])
        g_sc[...] = jnp.concatenate(
            [g, jnp.zeros((TB - B, H), jnp.float32)], axis=0).astype(jnp.bfloat16)

    ids = b_ref[...].reshape(1, RB)
    iota = lax.broadcasted_iota(jnp.int32, (TB, RB), 0)
    ohf = (ids == iota).astype(jnp.bfloat16)

    a = jnp.dot(x_ref[...], W1_ref[:F, :], preferred_element_type=jnp.float32)
    grow = lax.dot_general(ohf, g_sc[...], (((0,), (0,)), ((), ())),
                           preferred_element_type=jnp.float32)
    h = jax.nn.softplus(a + grow)
    srow = lax.dot_general(W2_ref[...], h, (((0,), (1,)), ((), ())),
                           preferred_element_type=jnp.float32)
    s_ref[...] = (srow + b2_ref[...])[0]


def _k3_scores(x, ids_pad, global_feat, W1, b1r, W2, b2r):
    return pl.pallas_call(
        _k3_body,
        grid=(NBLK,),
        in_specs=[
            pl.BlockSpec((RB, F), lambda i: (i, 0)),
            pl.BlockSpec((RB,), lambda i: (i,)),
            pl.BlockSpec((B, GFD), lambda i: (0, 0)),
            pl.BlockSpec((F + GFD, H), lambda i: (0, 0)),
            pl.BlockSpec((1, H), lambda i: (0, 0)),
            pl.BlockSpec((H, 1), lambda i: (0, 0)),
            pl.BlockSpec((1, 1), lambda i: (0, 0)),
        ],
        out_specs=pl.BlockSpec((RB,), lambda i: (i,)),
        out_shape=jax.ShapeDtypeStruct((NPAD,), jnp.float32),
        scratch_shapes=[
            pltpu.VMEM((TB, H), jnp.bfloat16),
        ],
    )(x, ids_pad, global_feat, W1, b1r, W2, b2r)


_SC_MESH = plsc.VectorSubcoreMesh(core_axis_name="c", subcore_axis_name="s")
_SC_PARAMS = pltpu.CompilerParams(needs_layout_passes=False)


@functools.partial(
    pl.kernel,
    out_type=jax.ShapeDtypeStruct((NPAD,), jnp.float32),
    mesh=_SC_MESH,
    scratch_types=[
        pltpu.VMEM((TPW,), jnp.float32),
        pltpu.VMEM((TPW,), jnp.int32),
        pltpu.VMEM((TPW,), jnp.float32),
        pltpu.VMEM((TB,), jnp.float32),
        pltpu.VMEM((TB,), jnp.float32),
        pltpu.VMEM((NS, TB), jnp.float32),
        pltpu.VMEM_SHARED((NS, TB), jnp.float32),
        pltpu.VMEM_SHARED((NS, TB), jnp.float32),
        pltpu.VMEM((L,), jnp.int32),
        pltpu.VMEM((L,), jnp.float32),
    ],
    compiler_params=_SC_PARAMS,
)
def _k4_softmax(s_hbm, ids_hbm, w_hbm,
                s_v, id_v, e_v, mtab, dtab, allv, shm, shd, bufi, bufv):
    cid = lax.axis_index("c")
    sid = lax.axis_index("s")

    @pl.when(cid == 0)
    def _core0():
        base = sid * TPW
        pltpu.sync_copy(s_hbm.at[pl.ds(base, TPW)], s_v)
        pltpu.sync_copy(ids_hbm.at[pl.ds(base, TPW)], id_v)

        lane = lax.iota(jnp.int32, L)
        shifts = [(k, jnp.maximum(lane - k, 0)) for k in (1, 2, 4, 8)]
        inx = jnp.minimum(lane + 1, L - 1)
        nvr = TPW // L

        for j in range(TB // L):
            mtab[pl.ds(j * L, L)] = jnp.full((L,), NEG_BIG, jnp.float32)

        def scan_combine(ids, val, op):
            bufi[...] = ids
            cur = val
            for k, idxk in shifts:
                ids_sh = plsc.load_gather(bufi, [idxk])
                bufv[...] = cur
                cur_sh = plsc.load_gather(bufv, [idxk])
                ok = jnp.logical_and(ids_sh == ids, lane >= k)
                cur = jnp.where(ok, op(cur, cur_sh), cur)
            ids_nx = plsc.load_gather(bufi, [inx])
            last = jnp.logical_or(ids_nx != ids, lane == L - 1)
            return cur, last

        def max_it(v, carry):
            off = v * L
            ids = id_v[pl.ds(off, L)]
            sv = s_v[pl.ds(off, L)]
            cur, last = scan_combine(ids, sv, jnp.maximum)
            mg = plsc.load_gather(mtab, [ids])
            plsc.store_scatter(mtab, [ids], jnp.maximum(mg, cur), mask=last)
            return carry

        lax.fori_loop(0, nvr, max_it, 0)

        pltpu.sync_copy(mtab, shm.at[sid])
        plsc.subcore_barrier()
        pltpu.sync_copy(shm, allv)

        def mred(j, carry):
            off = j * L
            acc = jnp.full((L,), NEG_BIG, jnp.float32)
            for r in range(NS):
                acc = jnp.maximum(acc, allv[r, pl.ds(off, L)])
            mtab[pl.ds(off, L)] = jnp.where(acc > NEG_BIG, acc, 0.0)
            dtab[pl.ds(off, L)] = jnp.zeros((L,), jnp.float32)
            return carry

        lax.fori_loop(0, TB // L, mred, 0)

        def sum_it(v, carry):
            off = v * L
            ids = id_v[pl.ds(off, L)]
            sv = s_v[pl.ds(off, L)]
            mg = plsc.load_gather(mtab, [ids])
            e = jnp.exp(sv - mg)
            e_v[pl.ds(off, L)] = e
            cur, last = scan_combine(ids, e, jnp.add)
            dg = plsc.load_gather(dtab, [ids])
            plsc.store_scatter(dtab, [ids], dg + cur, mask=last)
            return carry

        lax.fori_loop(0, nvr, sum_it, 0)

        pltpu.sync_copy(dtab, shd.at[sid])
        plsc.subcore_barrier()
        pltpu.sync_copy(shd, allv)

        def dred(j, carry):
            off = j * L
            acc = jnp.zeros((L,), jnp.float32)
            for r in range(NS):
                acc = acc + allv[r, pl.ds(off, L)]
            dtab[pl.ds(off, L)] = acc
            return carry

        lax.fori_loop(0, TB // L, dred, 0)

        def norm_it(v, carry):
            off = v * L
            ids = id_v[pl.ds(off, L)]
            dg = plsc.load_gather(dtab, [ids])
            e_v[pl.ds(off, L)] = e_v[pl.ds(off, L)] / (dg + 1e-16)
            return carry

        lax.fori_loop(0, nvr, norm_it, 0)

        pltpu.sync_copy(e_v, w_hbm.at[pl.ds(base, TPW)])


def kernel(x, batch, global_feat, W1, b1, W2, b2):
    batch_i32 = batch.astype(jnp.int32)
    ids_pad = jnp.pad(batch_i32, (0, NPAD - N), constant_values=B)
    b1r = b1.reshape(1, H).astype(jnp.float32)
    b2r = b2.reshape(1, 1).astype(jnp.float32)

    s = _k3_scores(x, ids_pad, global_feat, W1, b1r, W2, b2r)
    w_pad = _k4_softmax(s, ids_pad)
    return w_pad[:N].reshape(N, 1)


# speedup vs baseline: 8.1666x; 1.0627x over previous
import functools

import jax
import jax.numpy as jnp
from jax import lax
from jax.experimental import pallas as pl
from jax.experimental.pallas import tpu as pltpu
from jax.experimental.pallas import tpu_sc as plsc

N = 50000
B = 512
F = 256
H = 32
GFD = 103
TB = 528
NS = 16
NPAD = 50176
TPW = NPAD // NS
RB = 1024
NBLK = NPAD // RB
L = 16

NEG_BIG = -3.0e38


def _k3_body(x_ref, b_ref, gf_ref, W1_ref, b1_ref, W2_ref, b2_ref,
             s_ref, g_sc):
    i = pl.program_id(0)

    @pl.when(i == 0)
    def _init_g():
        g = (jnp.dot(gf_ref[...], W1_ref[F:, :],
                     preferred_element_type=jnp.float32) + b1_ref[...])
        g_sc[...] = jnp.concatenate(
            [g, jnp.zeros((TB - B, H), jnp.float32)], axis=0).astype(jnp.bfloat16)

    ids = b_ref[...].reshape(1, RB)
    iota = lax.broadcasted_iota(jnp.int32, (TB, RB), 0)
    ohf = (ids == iota).astype(jnp.bfloat16)

    a = jnp.dot(x_ref[...], W1_ref[:F, :], preferred_element_type=jnp.float32)
    grow = lax.dot_general(ohf, g_sc[...], (((0,), (0,)), ((), ())),
                           preferred_element_type=jnp.float32)
    h = jax.nn.softplus(a + grow)
    srow = lax.dot_general(W2_ref[...], h, (((0,), (1,)), ((), ())),
                           preferred_element_type=jnp.float32)
    s_ref[...] = (srow + b2_ref[...])[0]


def _k3_scores(x, ids_pad, global_feat, W1, b1r, W2, b2r):
    return pl.pallas_call(
        _k3_body,
        grid=(NBLK,),
        in_specs=[
            pl.BlockSpec((RB, F), lambda i: (i, 0)),
            pl.BlockSpec((RB,), lambda i: (i,)),
            pl.BlockSpec((B, GFD), lambda i: (0, 0)),
            pl.BlockSpec((F + GFD, H), lambda i: (0, 0)),
            pl.BlockSpec((1, H), lambda i: (0, 0)),
            pl.BlockSpec((H, 1), lambda i: (0, 0)),
            pl.BlockSpec((1, 1), lambda i: (0, 0)),
        ],
        out_specs=pl.BlockSpec((RB,), lambda i: (i,)),
        out_shape=jax.ShapeDtypeStruct((NPAD,), jnp.float32),
        scratch_shapes=[
            pltpu.VMEM((TB, H), jnp.bfloat16),
        ],
    )(x, ids_pad, global_feat, W1, b1r, W2, b2r)


_SC_MESH = plsc.VectorSubcoreMesh(core_axis_name="c", subcore_axis_name="s")
_SC_PARAMS = pltpu.CompilerParams(needs_layout_passes=False)


@functools.partial(
    pl.kernel,
    out_type=jax.ShapeDtypeStruct((NPAD,), jnp.float32),
    mesh=_SC_MESH,
    scratch_types=[
        pltpu.VMEM((TPW,), jnp.float32),
        pltpu.VMEM((TPW,), jnp.int32),
        pltpu.VMEM((TPW,), jnp.float32),
        pltpu.VMEM((TB,), jnp.float32),
        pltpu.VMEM((TB,), jnp.float32),
        pltpu.VMEM((NS, TB), jnp.float32),
        pltpu.VMEM_SHARED((NS, TB), jnp.float32),
        pltpu.VMEM_SHARED((NS, TB), jnp.float32),
    ],
    compiler_params=_SC_PARAMS,
)
def _k4_softmax(s_hbm, ids_hbm, w_hbm,
                s_v, id_v, e_v, mtab, dtab, allv, shm, shd):
    cid = lax.axis_index("c")
    sid = lax.axis_index("s")

    @pl.when(cid == 0)
    def _core0():
        base = sid * TPW
        pltpu.sync_copy(s_hbm.at[pl.ds(base, TPW)], s_v)
        pltpu.sync_copy(ids_hbm.at[pl.ds(base, TPW)], id_v)

        lane = lax.iota(jnp.int32, L)
        shifts = [(k, jnp.maximum(lane - k, 0)) for k in (1, 2, 4, 8)]
        inx = jnp.minimum(lane + 1, L - 1)
        nvr = TPW // L

        for j in range(TB // L):
            mtab[pl.ds(j * L, L)] = jnp.full((L,), NEG_BIG, jnp.float32)

        gdn = lax.GatherDimensionNumbers(
            offset_dims=(), collapsed_slice_dims=(0,), start_index_map=(0,))

        def shuf(vec, idx):
            return lax.gather(vec, idx[:, None], gdn, (1,),
                              mode=lax.GatherScatterMode.PROMISE_IN_BOUNDS)

        def scan_combine(ids, val, op):
            cur = val
            for k, idxk in shifts:
                ids_sh = shuf(ids, idxk)
                cur_sh = shuf(cur, idxk)
                ok = jnp.logical_and(ids_sh == ids, lane >= k)
                cur = jnp.where(ok, op(cur, cur_sh), cur)
            ids_nx = shuf(ids, inx)
            last = jnp.logical_or(ids_nx != ids, lane == L - 1)
            return cur, last

        def max_it(v, carry):
            off = v * L
            ids = id_v[pl.ds(off, L)]
            sv = s_v[pl.ds(off, L)]
            cur, last = scan_combine(ids, sv, jnp.maximum)
            mg = plsc.load_gather(mtab, [ids])
            plsc.store_scatter(mtab, [ids], jnp.maximum(mg, cur), mask=last)
            return carry

        lax.fori_loop(0, nvr, max_it, 0)

        pltpu.sync_copy(mtab, shm.at[sid])
        plsc.subcore_barrier()
        pltpu.sync_copy(shm, allv)

        def mred(j, carry):
            off = j * L
            acc = jnp.full((L,), NEG_BIG, jnp.float32)
            for r in range(NS):
                acc = jnp.maximum(acc, allv[r, pl.ds(off, L)])
            mtab[pl.ds(off, L)] = jnp.where(acc > NEG_BIG, acc, 0.0)
            dtab[pl.ds(off, L)] = jnp.zeros((L,), jnp.float32)
            return carry

        lax.fori_loop(0, TB // L, mred, 0)

        def sum_it(v, carry):
            off = v * L
            ids = id_v[pl.ds(off, L)]
            sv = s_v[pl.ds(off, L)]
            mg = plsc.load_gather(mtab, [ids])
            e = jnp.exp(sv - mg)
            e_v[pl.ds(off, L)] = e
            cur, last = scan_combine(ids, e, jnp.add)
            dg = plsc.load_gather(dtab, [ids])
            plsc.store_scatter(dtab, [ids], dg + cur, mask=last)
            return carry

        lax.fori_loop(0, nvr, sum_it, 0)

        pltpu.sync_copy(dtab, shd.at[sid])
        plsc.subcore_barrier()
        pltpu.sync_copy(shd, allv)

        def dred(j, carry):
            off = j * L
            acc = jnp.zeros((L,), jnp.float32)
            for r in range(NS):
                acc = acc + allv[r, pl.ds(off, L)]
            dtab[pl.ds(off, L)] = acc
            return carry

        lax.fori_loop(0, TB // L, dred, 0)

        def norm_it(v, carry):
            off = v * L
            ids = id_v[pl.ds(off, L)]
            dg = plsc.load_gather(dtab, [ids])
            e_v[pl.ds(off, L)] = e_v[pl.ds(off, L)] / (dg + 1e-16)
            return carry

        lax.fori_loop(0, nvr, norm_it, 0)

        pltpu.sync_copy(e_v, w_hbm.at[pl.ds(base, TPW)])


def kernel(x, batch, global_feat, W1, b1, W2, b2):
    batch_i32 = batch.astype(jnp.int32)
    ids_pad = jnp.pad(batch_i32, (0, NPAD - N), constant_values=B)
    b1r = b1.reshape(1, H).astype(jnp.float32)
    b2r = b2.reshape(1, 1).astype(jnp.float32)

    s = _k3_scores(x, ids_pad, global_feat, W1, b1r, W2, b2r)
    w_pad = _k4_softmax(s, ids_pad)
    return w_pad[:N].reshape(N, 1)


# speedup vs baseline: 8.9510x; 1.0961x over previous
import functools

import jax
import jax.numpy as jnp
from jax import lax
from jax.experimental import pallas as pl
from jax.experimental.pallas import tpu as pltpu
from jax.experimental.pallas import tpu_sc as plsc

N = 50000
B = 512
F = 256
H = 32
GFD = 103
TB = 528
NS = 16
NPAD = 51200
TPW = NPAD // NS
RB = 2048
NBLK = NPAD // RB
L = 16

NEG_BIG = -3.0e38


def _k3_body(x_ref, b_ref, gf_ref, W1_ref, b1_ref, W2_ref, b2_ref,
             s_ref, g_sc):
    i = pl.program_id(0)

    @pl.when(i == 0)
    def _init_g():
        g = (jnp.dot(gf_ref[...], W1_ref[F:, :],
                     preferred_element_type=jnp.float32) + b1_ref[...])
        g_sc[...] = g.astype(jnp.bfloat16)

    ids = jnp.minimum(b_ref[...].reshape(1, RB), B - 1)
    iota = lax.broadcasted_iota(jnp.int32, (B, RB), 0)
    ohf = (ids == iota).astype(jnp.bfloat16)

    a = jnp.dot(x_ref[...], W1_ref[:F, :], preferred_element_type=jnp.float32)
    grow = lax.dot_general(ohf, g_sc[...], (((0,), (0,)), ((), ())),
                           preferred_element_type=jnp.float32)
    h = jax.nn.softplus(a + grow)
    srow = lax.dot_general(W2_ref[...], h, (((0,), (1,)), ((), ())),
                           preferred_element_type=jnp.float32)
    s_ref[...] = (srow + b2_ref[...])[0]


def _k3_scores(x, ids_pad, global_feat, W1, b1r, W2, b2r):
    return pl.pallas_call(
        _k3_body,
        grid=(NBLK,),
        in_specs=[
            pl.BlockSpec((RB, F), lambda i: (i, 0)),
            pl.BlockSpec((RB,), lambda i: (i,)),
            pl.BlockSpec((B, GFD), lambda i: (0, 0)),
            pl.BlockSpec((F + GFD, H), lambda i: (0, 0)),
            pl.BlockSpec((1, H), lambda i: (0, 0)),
            pl.BlockSpec((H, 1), lambda i: (0, 0)),
            pl.BlockSpec((1, 1), lambda i: (0, 0)),
        ],
        out_specs=pl.BlockSpec((RB,), lambda i: (i,)),
        out_shape=jax.ShapeDtypeStruct((NPAD,), jnp.float32),
        scratch_shapes=[
            pltpu.VMEM((B, H), jnp.bfloat16),
        ],
    )(x, ids_pad, global_feat, W1, b1r, W2, b2r)


_SC_MESH = plsc.VectorSubcoreMesh(core_axis_name="c", subcore_axis_name="s")
_SC_PARAMS = pltpu.CompilerParams(needs_layout_passes=False)


@functools.partial(
    pl.kernel,
    out_type=jax.ShapeDtypeStruct((NPAD,), jnp.float32),
    mesh=_SC_MESH,
    scratch_types=[
        pltpu.VMEM((TPW,), jnp.float32),
        pltpu.VMEM((TPW,), jnp.int32),
        pltpu.VMEM((TPW,), jnp.float32),
        pltpu.VMEM((TB,), jnp.float32),
        pltpu.VMEM((TB,), jnp.float32),
        pltpu.VMEM((NS, TB), jnp.float32),
        pltpu.VMEM_SHARED((NS, TB), jnp.float32),
        pltpu.VMEM_SHARED((NS, TB), jnp.float32),
        pltpu.VMEM((L,), jnp.int32),
        pltpu.VMEM((L,), jnp.float32),
    ],
    compiler_params=_SC_PARAMS,
)
def _k4_softmax(s_hbm, ids_hbm, w_hbm,
                s_v, id_v, e_v, mtab, dtab, allv, shm, shd, bufi, bufv):
    cid = lax.axis_index("c")
    sid = lax.axis_index("s")

    @pl.when(cid == 0)
    def _core0():
        base = sid * TPW
        pltpu.sync_copy(s_hbm.at[pl.ds(base, TPW)], s_v)
        pltpu.sync_copy(ids_hbm.at[pl.ds(base, TPW)], id_v)

        nvr = TPW // L

        for j in range(TB // L):
            mtab[pl.ds(j * L, L)] = jnp.full((L,), NEG_BIG, jnp.float32)

        lane = lax.iota(jnp.int32, L)
        shifts = [(k, jnp.maximum(lane - k, 0)) for k in (1, 2, 4, 8)]
        inx = jnp.minimum(lane + 1, L - 1)

        def scan_combine(ids, val, op):
            bufi[...] = ids
            cur = val
            for k, idxk in shifts:
                ids_sh = plsc.load_gather(bufi, [idxk])
                bufv[...] = cur
                cur_sh = plsc.load_gather(bufv, [idxk])
                ok = jnp.logical_and(ids_sh == ids, lane >= k)
                cur = jnp.where(ok, op(cur, cur_sh), cur)
            ids_nx = plsc.load_gather(bufi, [inx])
            last = jnp.logical_or(ids_nx != ids, lane == L - 1)
            return cur, last

        def max_it(v, carry):
            off = v * L
            ids = id_v[pl.ds(off, L)]
            sv = s_v[pl.ds(off, L)]
            cur, last = scan_combine(ids, sv, jnp.maximum)
            mg = plsc.load_gather(mtab, [ids])
            plsc.store_scatter(mtab, [ids], jnp.maximum(mg, cur), mask=last)
            return carry

        lax.fori_loop(0, nvr, max_it, 0)

        pltpu.sync_copy(mtab, shm.at[sid])
        plsc.subcore_barrier()
        pltpu.sync_copy(shm, allv)

        def mred(j, carry):
            off = j * L
            acc = jnp.full((L,), NEG_BIG, jnp.float32)
            for r in range(NS):
                acc = jnp.maximum(acc, allv[r, pl.ds(off, L)])
            mtab[pl.ds(off, L)] = jnp.where(acc > NEG_BIG, acc, 0.0)
            dtab[pl.ds(off, L)] = jnp.zeros((L,), jnp.float32)
            return carry

        lax.fori_loop(0, TB // L, mred, 0)

        def sum_it(v, carry):
            off = v * L
            ids = id_v[pl.ds(off, L)]
            sv = s_v[pl.ds(off, L)]
            mg = plsc.load_gather(mtab, [ids])
            e = jnp.exp(sv - mg)
            e_v[pl.ds(off, L)] = e
            cur, last = scan_combine(ids, e, jnp.add)
            dg = plsc.load_gather(dtab, [ids])
            plsc.store_scatter(dtab, [ids], dg + cur, mask=last)
            return carry

        lax.fori_loop(0, nvr, sum_it, 0)

        pltpu.sync_copy(dtab, shd.at[sid])
        plsc.subcore_barrier()
        pltpu.sync_copy(shd, allv)

        def dred(j, carry):
            off = j * L
            acc = jnp.zeros((L,), jnp.float32)
            for r in range(NS):
                acc = acc + allv[r, pl.ds(off, L)]
            dtab[pl.ds(off, L)] = acc
            return carry

        lax.fori_loop(0, TB // L, dred, 0)

        def norm_it(v, carry):
            off = v * L
            ids = id_v[pl.ds(off, L)]
            dg = plsc.load_gather(dtab, [ids])
            e_v[pl.ds(off, L)] = e_v[pl.ds(off, L)] / (dg + 1e-16)
            return carry

        lax.fori_loop(0, nvr, norm_it, 0)

        pltpu.sync_copy(e_v, w_hbm.at[pl.ds(base, TPW)])


def kernel(x, batch, global_feat, W1, b1, W2, b2):
    batch_i32 = batch.astype(jnp.int32)
    ids_pad = jnp.pad(batch_i32, (0, NPAD - N), constant_values=B)
    b1r = b1.reshape(1, H).astype(jnp.float32)
    b2r = b2.reshape(1, 1).astype(jnp.float32)

    s = _k3_scores(x, ids_pad, global_feat, W1, b1r, W2, b2r)
    w_pad = _k4_softmax(s, ids_pad)
    return w_pad[:N].reshape(N, 1)


# speedup vs baseline: 9.0399x; 1.0099x over previous
import functools

import jax
import jax.numpy as jnp
from jax import lax
from jax.experimental import pallas as pl
from jax.experimental.pallas import tpu as pltpu
from jax.experimental.pallas import tpu_sc as plsc

N = 50000
B = 512
F = 256
H = 32
GFD = 103
TB = 528
NS = 16
NPAD = 53248
TPW = NPAD // NS
RB = 4096
NBLK = NPAD // RB
L = 16

NEG_BIG = -3.0e38


def _k3_body(x_ref, b_ref, gf_ref, W1_ref, b1_ref, W2_ref, b2_ref,
             s_ref, g_sc):
    i = pl.program_id(0)

    @pl.when(i == 0)
    def _init_g():
        g = (jnp.dot(gf_ref[...], W1_ref[F:, :],
                     preferred_element_type=jnp.float32) + b1_ref[...])
        g_sc[...] = g.astype(jnp.bfloat16)

    ids = jnp.minimum(b_ref[...].reshape(1, RB), B - 1)
    iota = lax.broadcasted_iota(jnp.int32, (B, RB), 0)
    ohf = (ids == iota).astype(jnp.bfloat16)

    a = jnp.dot(x_ref[...], W1_ref[:F, :], preferred_element_type=jnp.float32)
    grow = lax.dot_general(ohf, g_sc[...], (((0,), (0,)), ((), ())),
                           preferred_element_type=jnp.float32)
    h = jax.nn.softplus(a + grow)
    srow = lax.dot_general(W2_ref[...], h, (((0,), (1,)), ((), ())),
                           preferred_element_type=jnp.float32)
    s_ref[...] = (srow + b2_ref[...])[0]


def _k3_scores(x, ids_pad, global_feat, W1, b1r, W2, b2r):
    return pl.pallas_call(
        _k3_body,
        grid=(NBLK,),
        in_specs=[
            pl.BlockSpec((RB, F), lambda i: (i, 0)),
            pl.BlockSpec((RB,), lambda i: (i,)),
            pl.BlockSpec((B, GFD), lambda i: (0, 0)),
            pl.BlockSpec((F + GFD, H), lambda i: (0, 0)),
            pl.BlockSpec((1, H), lambda i: (0, 0)),
            pl.BlockSpec((H, 1), lambda i: (0, 0)),
            pl.BlockSpec((1, 1), lambda i: (0, 0)),
        ],
        out_specs=pl.BlockSpec((RB,), lambda i: (i,)),
        out_shape=jax.ShapeDtypeStruct((NPAD,), jnp.float32),
        scratch_shapes=[
            pltpu.VMEM((B, H), jnp.bfloat16),
        ],
    )(x, ids_pad, global_feat, W1, b1r, W2, b2r)


_SC_MESH = plsc.VectorSubcoreMesh(core_axis_name="c", subcore_axis_name="s")
_SC_PARAMS = pltpu.CompilerParams(needs_layout_passes=False)


@functools.partial(
    pl.kernel,
    out_type=jax.ShapeDtypeStruct((NPAD,), jnp.float32),
    mesh=_SC_MESH,
    scratch_types=[
        pltpu.VMEM((TPW,), jnp.float32),
        pltpu.VMEM((TPW,), jnp.int32),
        pltpu.VMEM((TPW,), jnp.float32),
        pltpu.VMEM((TB,), jnp.float32),
        pltpu.VMEM((TB,), jnp.float32),
        pltpu.VMEM((NS, TB), jnp.float32),
        pltpu.VMEM_SHARED((NS, TB), jnp.float32),
        pltpu.VMEM_SHARED((NS, TB), jnp.float32),
        pltpu.VMEM((L,), jnp.int32),
        pltpu.VMEM((L,), jnp.float32),
    ],
    compiler_params=_SC_PARAMS,
)
def _k4_softmax(s_hbm, ids_hbm, w_hbm,
                s_v, id_v, e_v, mtab, dtab, allv, shm, shd, bufi, bufv):
    cid = lax.axis_index("c")
    sid = lax.axis_index("s")

    @pl.when(cid == 0)
    def _core0():
        base = sid * TPW
        pltpu.sync_copy(s_hbm.at[pl.ds(base, TPW)], s_v)
        pltpu.sync_copy(ids_hbm.at[pl.ds(base, TPW)], id_v)

        nvr = TPW // L

        for j in range(TB // L):
            mtab[pl.ds(j * L, L)] = jnp.full((L,), NEG_BIG, jnp.float32)

        lane = lax.iota(jnp.int32, L)
        shifts = [(k, jnp.maximum(lane - k, 0)) for k in (1, 2, 4, 8)]
        inx = jnp.minimum(lane + 1, L - 1)

        def scan_combine(ids, val, op):
            bufi[...] = ids
            cur = val
            for k, idxk in shifts:
                ids_sh = plsc.load_gather(bufi, [idxk])
                bufv[...] = cur
                cur_sh = plsc.load_gather(bufv, [idxk])
                ok = jnp.logical_and(ids_sh == ids, lane >= k)
                cur = jnp.where(ok, op(cur, cur_sh), cur)
            ids_nx = plsc.load_gather(bufi, [inx])
            last = jnp.logical_or(ids_nx != ids, lane == L - 1)
            return cur, last

        def max_it(v, carry):
            off = v * L
            ids = id_v[pl.ds(off, L)]
            sv = s_v[pl.ds(off, L)]
            cur, last = scan_combine(ids, sv, jnp.maximum)
            mg = plsc.load_gather(mtab, [ids])
            plsc.store_scatter(mtab, [ids], jnp.maximum(mg, cur), mask=last)
            return carry

        lax.fori_loop(0, nvr, max_it, 0)

        pltpu.sync_copy(mtab, shm.at[sid])
        plsc.subcore_barrier()
        pltpu.sync_copy(shm, allv)

        def mred(j, carry):
            off = j * L
            acc = jnp.full((L,), NEG_BIG, jnp.float32)
            for r in range(NS):
                acc = jnp.maximum(acc, allv[r, pl.ds(off, L)])
            mtab[pl.ds(off, L)] = jnp.where(acc > NEG_BIG, acc, 0.0)
            dtab[pl.ds(off, L)] = jnp.zeros((L,), jnp.float32)
            return carry

        lax.fori_loop(0, TB // L, mred, 0)

        def sum_it(v, carry):
            off = v * L
            ids = id_v[pl.ds(off, L)]
            sv = s_v[pl.ds(off, L)]
            mg = plsc.load_gather(mtab, [ids])
            e = jnp.exp(sv - mg)
            e_v[pl.ds(off, L)] = e
            cur, last = scan_combine(ids, e, jnp.add)
            dg = plsc.load_gather(dtab, [ids])
            plsc.store_scatter(dtab, [ids], dg + cur, mask=last)
            return carry

        lax.fori_loop(0, nvr, sum_it, 0)

        pltpu.sync_copy(dtab, shd.at[sid])
        plsc.subcore_barrier()
        pltpu.sync_copy(shd, allv)

        def dred(j, carry):
            off = j * L
            acc = jnp.zeros((L,), jnp.float32)
            for r in range(NS):
                acc = acc + allv[r, pl.ds(off, L)]
            dtab[pl.ds(off, L)] = acc
            return carry

        lax.fori_loop(0, TB // L, dred, 0)

        def norm_it(v, carry):
            off = v * L
            ids = id_v[pl.ds(off, L)]
            dg = plsc.load_gather(dtab, [ids])
            e_v[pl.ds(off, L)] = e_v[pl.ds(off, L)] / (dg + 1e-16)
            return carry

        lax.fori_loop(0, nvr, norm_it, 0)

        pltpu.sync_copy(e_v, w_hbm.at[pl.ds(base, TPW)])


def kernel(x, batch, global_feat, W1, b1, W2, b2):
    batch_i32 = batch.astype(jnp.int32)
    ids_pad = jnp.pad(batch_i32, (0, NPAD - N), constant_values=B)
    b1r = b1.reshape(1, H).astype(jnp.float32)
    b2r = b2.reshape(1, 1).astype(jnp.float32)

    s = _k3_scores(x, ids_pad, global_feat, W1, b1r, W2, b2r)
    w_pad = _k4_softmax(s, ids_pad)
    return w_pad[:N].reshape(N, 1)


# speedup vs baseline: 9.2262x; 1.0206x over previous
import functools

import jax
import jax.numpy as jnp
from jax import lax
from jax.experimental import pallas as pl
from jax.experimental.pallas import tpu as pltpu
from jax.experimental.pallas import tpu_sc as plsc

N = 50000
B = 512
F = 256
H = 32
GFD = 103
TB = 528
NS = 16
NPAD = 53248
TPW = NPAD // NS
RB = 4096
NBLK = NPAD // RB
L = 16

NEG_BIG = -3.0e38


def _k3_body(x_ref, b_ref, gf_ref, W1_ref, b1_ref, W2_ref, b2_ref,
             s_ref, g_sc):
    i = pl.program_id(0)

    @pl.when(i == 0)
    def _init_g():
        g = (jnp.dot(gf_ref[...], W1_ref[F:, :],
                     preferred_element_type=jnp.float32) + b1_ref[...])
        g_sc[...] = g.astype(jnp.bfloat16)

    ids = jnp.minimum(b_ref[...].reshape(1, RB), B - 1)
    iota = lax.broadcasted_iota(jnp.int32, (B, RB), 0)
    ohf = (ids == iota).astype(jnp.bfloat16)

    a = jnp.dot(x_ref[...], W1_ref[:F, :], preferred_element_type=jnp.float32)
    grow = lax.dot_general(ohf, g_sc[...], (((0,), (0,)), ((), ())),
                           preferred_element_type=jnp.float32)
    h = jax.nn.softplus(a + grow)
    srow = lax.dot_general(W2_ref[...], h, (((0,), (1,)), ((), ())),
                           preferred_element_type=jnp.float32)
    s_ref[...] = (srow + b2_ref[...])[0]


def _k3_scores(x, ids_pad, global_feat, W1, b1r, W2, b2r):
    return pl.pallas_call(
        _k3_body,
        grid=(NBLK,),
        in_specs=[
            pl.BlockSpec((RB, F), lambda i: (i, 0)),
            pl.BlockSpec((RB,), lambda i: (i,)),
            pl.BlockSpec((B, GFD), lambda i: (0, 0)),
            pl.BlockSpec((F + GFD, H), lambda i: (0, 0)),
            pl.BlockSpec((1, H), lambda i: (0, 0)),
            pl.BlockSpec((H, 1), lambda i: (0, 0)),
            pl.BlockSpec((1, 1), lambda i: (0, 0)),
        ],
        out_specs=pl.BlockSpec((RB,), lambda i: (i,)),
        out_shape=jax.ShapeDtypeStruct((NPAD,), jnp.float32),
        scratch_shapes=[
            pltpu.VMEM((B, H), jnp.bfloat16),
        ],
    )(x, ids_pad, global_feat, W1, b1r, W2, b2r)


_SC_MESH = plsc.VectorSubcoreMesh(core_axis_name="c", subcore_axis_name="s")
_SC_PARAMS = pltpu.CompilerParams(needs_layout_passes=False)


@functools.partial(
    pl.kernel,
    out_type=jax.ShapeDtypeStruct((NPAD,), jnp.float32),
    mesh=_SC_MESH,
    scratch_types=[
        pltpu.VMEM((TPW,), jnp.float32),
        pltpu.VMEM((TPW,), jnp.int32),
        pltpu.VMEM((TPW,), jnp.float32),
        pltpu.VMEM((TB,), jnp.float32),
        pltpu.VMEM((TB,), jnp.float32),
        pltpu.VMEM((NS, TB), jnp.float32),
        pltpu.VMEM_SHARED((NS, TB), jnp.float32),
        pltpu.VMEM_SHARED((NS, TB), jnp.float32),
        pltpu.VMEM((L,), jnp.int32),
        pltpu.VMEM((L,), jnp.float32),
        pltpu.VMEM((L,), jnp.int32),
        pltpu.VMEM((L,), jnp.float32),
    ],
    compiler_params=_SC_PARAMS,
)
def _k4_softmax(s_hbm, ids_hbm, w_hbm,
                s_v, id_v, e_v, mtab, dtab, allv, shm, shd,
                bufi, bufv, bufi2, bufv2):
    cid = lax.axis_index("c")
    sid = lax.axis_index("s")

    @pl.when(cid == 0)
    def _core0():
        base = sid * TPW
        pltpu.sync_copy(s_hbm.at[pl.ds(base, TPW)], s_v)
        pltpu.sync_copy(ids_hbm.at[pl.ds(base, TPW)], id_v)

        nvr = TPW // L

        for j in range(TB // L):
            mtab[pl.ds(j * L, L)] = jnp.full((L,), NEG_BIG, jnp.float32)

        lane = lax.iota(jnp.int32, L)
        shifts = [(k, jnp.maximum(lane - k, 0)) for k in (1, 2, 4, 8)]
        inx = jnp.minimum(lane + 1, L - 1)

        def scan_combine(ids, val, op, bi, bv):
            bi[...] = ids
            cur = val
            for k, idxk in shifts:
                ids_sh = plsc.load_gather(bi, [idxk])
                bv[...] = cur
                cur_sh = plsc.load_gather(bv, [idxk])
                ok = jnp.logical_and(ids_sh == ids, lane >= k)
                cur = jnp.where(ok, op(cur, cur_sh), cur)
            ids_nx = plsc.load_gather(bi, [inx])
            last = jnp.logical_or(ids_nx != ids, lane == L - 1)
            return cur, last

        bufs = ((bufi, bufv), (bufi2, bufv2))

        def max_it(v, carry):
            for u, (bi, bv) in enumerate(bufs):
                off = (v * 2 + u) * L
                ids = id_v[pl.ds(off, L)]
                sv = s_v[pl.ds(off, L)]
                cur, last = scan_combine(ids, sv, jnp.maximum, bi, bv)
                mg = plsc.load_gather(mtab, [ids])
                plsc.store_scatter(mtab, [ids], jnp.maximum(mg, cur), mask=last)
            return carry

        lax.fori_loop(0, nvr // 2, max_it, 0)

        pltpu.sync_copy(mtab, shm.at[sid])
        plsc.subcore_barrier()
        pltpu.sync_copy(shm, allv)

        def mred(j, carry):
            off = j * L
            acc = jnp.full((L,), NEG_BIG, jnp.float32)
            for r in range(NS):
                acc = jnp.maximum(acc, allv[r, pl.ds(off, L)])
            mtab[pl.ds(off, L)] = jnp.where(acc > NEG_BIG, acc, 0.0)
            dtab[pl.ds(off, L)] = jnp.zeros((L,), jnp.float32)
            return carry

        lax.fori_loop(0, TB // L, mred, 0)

        def sum_it(v, carry):
            for u, (bi, bv) in enumerate(bufs):
                off = (v * 2 + u) * L
                ids = id_v[pl.ds(off, L)]
                sv = s_v[pl.ds(off, L)]
                mg = plsc.load_gather(mtab, [ids])
                e = jnp.exp(sv - mg)
                e_v[pl.ds(off, L)] = e
                cur, last = scan_combine(ids, e, jnp.add, bi, bv)
                dg = plsc.load_gather(dtab, [ids])
                plsc.store_scatter(dtab, [ids], dg + cur, mask=last)
            return carry

        lax.fori_loop(0, nvr // 2, sum_it, 0)

        pltpu.sync_copy(dtab, shd.at[sid])
        plsc.subcore_barrier()
        pltpu.sync_copy(shd, allv)

        def dred(j, carry):
            off = j * L
            acc = jnp.zeros((L,), jnp.float32)
            for r in range(NS):
                acc = acc + allv[r, pl.ds(off, L)]
            dtab[pl.ds(off, L)] = acc
            return carry

        lax.fori_loop(0, TB // L, dred, 0)

        def norm_it(v, carry):
            for u in range(4):
                off = (v * 4 + u) * L
                ids = id_v[pl.ds(off, L)]
                dg = plsc.load_gather(dtab, [ids])
                e_v[pl.ds(off, L)] = e_v[pl.ds(off, L)] / (dg + 1e-16)
            return carry

        lax.fori_loop(0, nvr // 4, norm_it, 0)

        pltpu.sync_copy(e_v, w_hbm.at[pl.ds(base, TPW)])


def kernel(x, batch, global_feat, W1, b1, W2, b2):
    batch_i32 = batch.astype(jnp.int32)
    ids_pad = jnp.pad(batch_i32, (0, NPAD - N), constant_values=B)
    b1r = b1.reshape(1, H).astype(jnp.float32)
    b2r = b2.reshape(1, 1).astype(jnp.float32)

    s = _k3_scores(x, ids_pad, global_feat, W1, b1r, W2, b2r)
    w_pad = _k4_softmax(s, ids_pad)
    return w_pad[:N].reshape(N, 1)
